# baseline (speedup 1.0000x reference)
import functools

import jax
import jax.numpy as jnp
from jax.experimental import pallas as pl
from jax.experimental.pallas import tpu as pltpu


def _fused_kernel(x_full_ref, adj_a_ref, adj_b_ref, w_ref, wr_ref, x_blk_ref,
                  out_ref, u_ref):
    i = pl.program_id(0)

    @pl.when(i == 0)
    def _():
        u_ref[...] = jnp.dot(
            x_full_ref[...], w_ref[...], preferred_element_type=jnp.float32
        )

    u = u_ref[...]
    r = jnp.dot(x_blk_ref[...], wr_ref[...], preferred_element_type=jnp.float32)
    bm = adj_a_ref.shape[0]
    out_ref[0:bm, :] = (
        jnp.dot(adj_a_ref[...], u, preferred_element_type=jnp.float32)
        + r[0:bm, :]
    )
    out_ref[bm : 2 * bm, :] = (
        jnp.dot(adj_b_ref[...], u, preferred_element_type=jnp.float32)
        + r[bm : 2 * bm, :]
    )


@jax.jit
def kernel(input, adj, h0, weight, weight_r):
    n, d = input.shape
    bm = 200 if n % 400 == 0 else n
    grid = (n // (2 * bm),)
    return pl.pallas_call(
        _fused_kernel,
        grid=grid,
        in_specs=[
            pl.BlockSpec((n, d), lambda i: (0, 0)),
            pl.BlockSpec((bm, n), lambda i: (2 * i, 0)),
            pl.BlockSpec((bm, n), lambda i: (2 * i + 1, 0)),
            pl.BlockSpec((d, d), lambda i: (0, 0)),
            pl.BlockSpec((d, d), lambda i: (0, 0)),
            pl.BlockSpec((2 * bm, d), lambda i: (i, 0)),
        ],
        out_specs=pl.BlockSpec((2 * bm, d), lambda i: (i, 0)),
        out_shape=jax.ShapeDtypeStruct((n, d), jnp.float32),
        scratch_shapes=[pltpu.VMEM((n, d), jnp.float32)],
        compiler_params=pltpu.CompilerParams(
            dimension_semantics=("arbitrary",),
        ),
    )(input, adj, adj, weight, weight_r, input)
